# baseline (speedup 1.0000x reference)
import functools

import jax
import jax.numpy as jnp
from jax.experimental import pallas as pl

_NUM_EMB = 8192
_HALF = 4096
_EMB_DIM = 32
_IN_DIM = 384
_COMMIT = 0.25
_BLK = 256


def _vq_body(lat_ref, w_ref, b_ref, embT_ref, emb_ref, s2_ref,
             z_ref, idx_ref, qst_ref, lsum_ref):
    i = pl.program_id(0)
    x = lat_ref[...]
    w = w_ref[...]
    z = jnp.dot(x, w) + b_ref[...]
    z_ref[...] = z
    s1 = jnp.sum(z * z, axis=1, keepdims=True)
    zb = z.astype(jnp.bfloat16)
    m = jnp.dot(zb, embT_ref[...], preferred_element_type=jnp.float32)
    dist = (s1 - 2.0 * m) + s2_ref[...]
    cols = jax.lax.broadcasted_iota(jnp.int32, (_BLK, _HALF), 1)
    d1 = dist[:, :_HALF]
    d2 = dist[:, _HALF:]
    v1 = jnp.min(d1, axis=1, keepdims=True)
    i1 = jnp.min(jnp.where(d1 == v1, cols, _NUM_EMB), axis=1)
    v2 = jnp.min(d2, axis=1, keepdims=True)
    i2 = jnp.min(jnp.where(d2 == v2, cols + _HALF, _NUM_EMB), axis=1)
    v1b = v1.astype(jnp.bfloat16).astype(jnp.float32)
    take2 = v2 < v1b
    idx = jnp.where(take2[:, 0], i2, i1)
    idx_ref[...] = idx
    allcols = jax.lax.broadcasted_iota(jnp.int32, (_BLK, _NUM_EMB), 1)
    onehot = (allcols == idx[:, None]).astype(jnp.float32)
    q = jnp.dot(onehot, emb_ref[...])
    qst_ref[...] = z + (q - z)
    diff = q - z
    part = jnp.sum(diff * diff).reshape(1, 1)

    @pl.when(i == 0)
    def _():
        lsum_ref[...] = jnp.zeros_like(lsum_ref)

    lsum_ref[...] += part


@functools.partial(jax.jit, static_argnums=())
def kernel(latents, W_enc, b_enc, emb):
    B, T, _ = latents.shape
    n_tok = B * T
    lat2 = latents.reshape(n_tok, _IN_DIM)
    b2 = b_enc.reshape(1, _EMB_DIM)
    embT16 = emb.T.astype(jnp.bfloat16)
    s2 = jnp.sum(emb ** 2, axis=1).reshape(1, _NUM_EMB)
    grid = (n_tok // _BLK,)
    z_flat, idx_flat, qst_flat, lsum = pl.pallas_call(
        _vq_body,
        grid=grid,
        in_specs=[
            pl.BlockSpec((_BLK, _IN_DIM), lambda i: (i, 0)),
            pl.BlockSpec((_IN_DIM, _EMB_DIM), lambda i: (0, 0)),
            pl.BlockSpec((1, _EMB_DIM), lambda i: (0, 0)),
            pl.BlockSpec((_EMB_DIM, _NUM_EMB), lambda i: (0, 0)),
            pl.BlockSpec((_NUM_EMB, _EMB_DIM), lambda i: (0, 0)),
            pl.BlockSpec((1, _NUM_EMB), lambda i: (0, 0)),
        ],
        out_specs=[
            pl.BlockSpec((_BLK, _EMB_DIM), lambda i: (i, 0)),
            pl.BlockSpec((_BLK,), lambda i: (i,)),
            pl.BlockSpec((_BLK, _EMB_DIM), lambda i: (i, 0)),
            pl.BlockSpec((1, 1), lambda i: (0, 0)),
        ],
        out_shape=[
            jax.ShapeDtypeStruct((n_tok, _EMB_DIM), jnp.float32),
            jax.ShapeDtypeStruct((n_tok,), jnp.int32),
            jax.ShapeDtypeStruct((n_tok, _EMB_DIM), jnp.float32),
            jax.ShapeDtypeStruct((1, 1), jnp.float32),
        ],
    )(lat2, W_enc, b2, embT16, emb, s2)
    c = lsum[0, 0] / jnp.float32(n_tok * _EMB_DIM)
    loss = c + jnp.float32(_COMMIT) * c
    quantized_st = qst_flat.reshape(B, T, _EMB_DIM)
    z_e = z_flat.reshape(B, T, _EMB_DIM)
    idx_out = idx_flat.reshape(B, T)
    return quantized_st, loss, idx_out, z_e
